# Initial kernel scaffold; baseline (speedup 1.0000x reference)
import jax
import jax.numpy as jnp
from jax.experimental import pallas as pl


def kernel(sent_vecs, concepts, adj, adj_lengths, emb, Wt, bt, gnn_w0, gnn_w1, wq, bq, wk, bk, wv, bv, fc_w0, fc_b0, ln_g, ln_b, fc_w1, fc_b1):
    raise NotImplementedError("write your pallas kernel here")


# speedup vs baseline: 2.0436x; 2.0436x over previous
import functools

import jax
import jax.numpy as jnp
from jax import lax
from jax.experimental import pallas as pl
from jax.experimental.pallas import tpu as pltpu
from jax.experimental.pallas import tpu_sc as plsc

_BS, _N_NODE, _N_REL = 32, 200, 34
_C_DIM = 100
_D_K = 50
_SENT_DIM = 1024
_FC_DIM = 200


def _gelu(x):
    return 0.5 * x * (1.0 + lax.erf(x * 0.7071067811865476))


def _sc_gather(emb, idx_flat):
    B, D = idx_flat.shape[0], emb.shape[1]
    NW = 32
    b_per_w = B // NW
    CH = 40
    n_ch = b_per_w // CH
    mesh = plsc.VectorSubcoreMesh(core_axis_name="c", subcore_axis_name="s")

    @functools.partial(
        pl.kernel,
        mesh=mesh,
        out_type=jax.ShapeDtypeStruct((B, D), jnp.float32),
        scratch_types=[
            pltpu.VMEM((b_per_w,), jnp.int32),
            pltpu.VMEM((CH, D), jnp.float32),
            pltpu.VMEM((CH, D), jnp.float32),
            pltpu.SemaphoreType.DMA,
            pltpu.SemaphoreType.DMA,
        ],
    )
    def k(table_hbm, idx_hbm, out_hbm, idx_v, rows0, rows1, sem0, sem1):
        wid = lax.axis_index("s") * 2 + lax.axis_index("c")
        base = wid * b_per_w
        pltpu.sync_copy(idx_hbm.at[pl.ds(base, b_per_w)], idx_v)
        bufs = (rows0, rows1)
        sems = (sem0, sem1)
        pending = pltpu.async_copy(
            table_hbm.at[idx_v.at[pl.ds(0, CH)]], bufs[0], sems[0])
        for c in range(n_ch):
            nxt = c + 1
            nxt_pending = None
            if nxt < n_ch:
                nxt_pending = pltpu.async_copy(
                    table_hbm.at[idx_v.at[pl.ds(nxt * CH, CH)]],
                    bufs[nxt % 2], sems[nxt % 2])
            pending.wait()
            pltpu.sync_copy(bufs[c % 2], out_hbm.at[pl.ds(base + c * CH, CH)])
            pending = nxt_pending

    return k(emb, idx_flat)


def _tc_body(gat_ref, adj_ref, sent_ref, len_ref,
             wt_ref, bt_ref, w0t_ref, w1t_ref,
             wq_ref, bq_ref, wkt_ref, bk_ref, wvt_ref, bv_ref,
             fcw_p0_ref, fcw_p1_ref, fcw_s_ref, fcb0_ref, lng_ref, lnb_ref,
             fcw1_ref, fcb1_ref,
             attn_ref, logit_ref):
    f32 = jnp.float32

    g = gat_ref[0]
    x0 = _gelu(jnp.dot(g, wt_ref[...], preferred_element_type=f32)
               + bt_ref[...])
    out_t = x0.T

    ones_row = jnp.ones((1, _N_NODE), dtype=f32)

    def rgcn_layer(out_t, w_ref):
        def body(r, acc):
            a_r = adj_ref[0, r]
            w_r = w_ref[r]
            h_t = jnp.dot(w_r, out_t, preferred_element_type=f32)
            m_t = jnp.dot(h_t, a_r, preferred_element_type=f32)
            deg = jnp.maximum(
                jnp.sum(a_r, axis=0, keepdims=True), 1.0)
            return acc + m_t / deg
        acc0 = jnp.zeros((_C_DIM, _N_NODE), dtype=f32)
        acc = lax.fori_loop(0, _N_REL, body, acc0)
        return _gelu(acc)

    out_t = rgcn_layer(out_t, w0t_ref)
    out_t = rgcn_layer(out_t, w1t_ref)

    sent = sent_ref[0]
    q = jnp.dot(sent, wq_ref[...], preferred_element_type=f32) + bq_ref[...]
    kk_t = jnp.dot(wkt_ref[...], out_t, preferred_element_type=f32) \
        + bk_ref[...]
    vv_t = jnp.dot(wvt_ref[...], out_t, preferred_element_type=f32) \
        + bv_ref[...]

    length = len_ref[0, 0, 0]
    node_ids = lax.broadcasted_iota(jnp.int32, (1, _N_NODE), 1)
    masked = node_ids >= length

    pooled = []
    attn_rows = []
    for h in range(2):
        q_h = q[:, h * _D_K:(h + 1) * _D_K]
        k_h = kk_t[h * _D_K:(h + 1) * _D_K, :]
        v_h = vv_t[h * _D_K:(h + 1) * _D_K, :]
        score = jnp.dot(q_h, k_h, preferred_element_type=f32) \
            * (1.0 / 7.0710678118654755)
        score = jnp.where(masked, -1e30, score)
        mx = jnp.max(score, axis=1, keepdims=True)
        e = jnp.exp(score - mx)
        attn = e / jnp.sum(e, axis=1, keepdims=True)
        attn_rows.append(attn)
        pooled.append(lax.dot_general(
            attn, v_h, (((1,), (1,)), ((), ())),
            preferred_element_type=f32))

    attn_ref[0] = jnp.concatenate(attn_rows, axis=0)

    f = (jnp.dot(pooled[0], fcw_p0_ref[...], preferred_element_type=f32)
         + jnp.dot(pooled[1], fcw_p1_ref[...], preferred_element_type=f32)
         + jnp.dot(sent, fcw_s_ref[...], preferred_element_type=f32)
         + fcb0_ref[...])
    mu = jnp.mean(f, axis=1, keepdims=True)
    d = f - mu
    v = jnp.mean(d * d, axis=1, keepdims=True)
    y = d * lax.rsqrt(v + 1e-5) * lng_ref[...] + lnb_ref[...]
    hh = _gelu(y)
    logit_ref[0] = jnp.dot(hh, fcw1_ref[...], preferred_element_type=f32) \
        + fcb1_ref[...]


def kernel(sent_vecs, concepts, adj, adj_lengths, emb, Wt, bt, gnn_w0, gnn_w1,
           wq, bq, wk, bk, wv, bv, fc_w0, fc_b0, ln_g, ln_b, fc_w1, fc_b1):
    f32 = jnp.float32
    bs = sent_vecs.shape[0]

    idx_flat = concepts.reshape(-1).astype(jnp.int32)
    gathered = _sc_gather(emb, idx_flat).reshape(bs, _N_NODE, 1024)

    w0t = jnp.transpose(gnn_w0.reshape(_C_DIM, _C_DIM, _N_REL), (2, 1, 0))
    w1t = jnp.transpose(gnn_w1.reshape(_C_DIM, _C_DIM, _N_REL), (2, 1, 0))
    sent3 = sent_vecs.reshape(bs, 1, _SENT_DIM)
    len3 = adj_lengths.reshape(bs, 1, 1).astype(jnp.int32)
    bt_row = bt.reshape(1, _C_DIM)
    bq_row = bq.reshape(1, _C_DIM)
    wkt = wk.T
    wvt = wv.T
    bk_col = bk.reshape(_C_DIM, 1)
    bv_col = bv.reshape(_C_DIM, 1)
    fcw_p0 = fc_w0[:_D_K]
    fcw_p1 = fc_w0[_D_K:2 * _D_K]
    fcw_s = fc_w0[2 * _D_K:]
    fcb0_row = fc_b0.reshape(1, _FC_DIM)
    lng_row = ln_g.reshape(1, _FC_DIM)
    lnb_row = ln_b.reshape(1, _FC_DIM)
    fcb1_row = fc_b1.reshape(1, 1)

    full = lambda *shape: pl.BlockSpec(shape, lambda b: (0,) * len(shape))
    attn_o, logit_o = pl.pallas_call(
        _tc_body,
        grid=(bs,),
        in_specs=[
            pl.BlockSpec((1, _N_NODE, 1024), lambda b: (b, 0, 0)),
            pl.BlockSpec((1, _N_REL, _N_NODE, _N_NODE),
                         lambda b: (b, 0, 0, 0)),
            pl.BlockSpec((1, 1, _SENT_DIM), lambda b: (b, 0, 0)),
            pl.BlockSpec((1, 1, 1), lambda b: (b, 0, 0)),
            full(1024, _C_DIM),
            full(1, _C_DIM),
            full(_N_REL, _C_DIM, _C_DIM),
            full(_N_REL, _C_DIM, _C_DIM),
            full(_SENT_DIM, _C_DIM),
            full(1, _C_DIM),
            full(_C_DIM, _C_DIM),
            full(_C_DIM, 1),
            full(_C_DIM, _C_DIM),
            full(_C_DIM, 1),
            full(_D_K, _FC_DIM),
            full(_D_K, _FC_DIM),
            full(_SENT_DIM, _FC_DIM),
            full(1, _FC_DIM),
            full(1, _FC_DIM),
            full(1, _FC_DIM),
            full(_FC_DIM, 1),
            full(1, 1),
        ],
        out_specs=[
            pl.BlockSpec((1, 2, _N_NODE), lambda b: (b, 0, 0)),
            pl.BlockSpec((1, 1, 1), lambda b: (b, 0, 0)),
        ],
        out_shape=[
            jax.ShapeDtypeStruct((bs, 2, _N_NODE), f32),
            jax.ShapeDtypeStruct((bs, 1, 1), f32),
        ],
    )(gathered, adj, sent3, len3, Wt, bt_row, w0t, w1t,
      wq, bq_row, wkt, bk_col, wvt, bv_col,
      fcw_p0, fcw_p1, fcw_s, fcb0_row, lng_row, lnb_row, fc_w1, fcb1_row)

    attn = jnp.transpose(attn_o, (1, 0, 2)).reshape(2 * bs, _N_NODE)
    logits = logit_o.reshape(bs, 1)
    return logits, attn


# speedup vs baseline: 4.8664x; 2.3813x over previous
import functools

import jax
import jax.numpy as jnp
from jax import lax
from jax.experimental import pallas as pl
from jax.experimental.pallas import tpu as pltpu
from jax.experimental.pallas import tpu_sc as plsc

_BS, _N_NODE, _N_REL = 32, 200, 34
_C_DIM = 100
_D_K = 50
_SENT_DIM = 1024
_FC_DIM = 200


def _gelu(x):
    return 0.5 * x * (1.0 + lax.erf(x * 0.7071067811865476))


def _sc_gather(emb, idx_flat):
    B, D = idx_flat.shape[0], emb.shape[1]
    NW = 32
    b_per_w = B // NW
    CH = 40
    n_ch = b_per_w // CH
    mesh = plsc.VectorSubcoreMesh(core_axis_name="c", subcore_axis_name="s")

    @functools.partial(
        pl.kernel,
        mesh=mesh,
        out_type=jax.ShapeDtypeStruct((B, D), jnp.float32),
        scratch_types=[
            pltpu.VMEM((b_per_w,), jnp.int32),
            pltpu.VMEM((CH, D), jnp.float32),
            pltpu.VMEM((CH, D), jnp.float32),
            pltpu.SemaphoreType.DMA,
            pltpu.SemaphoreType.DMA,
        ],
    )
    def k(table_hbm, idx_hbm, out_hbm, idx_v, rows0, rows1, sem0, sem1):
        wid = lax.axis_index("s") * 2 + lax.axis_index("c")
        base = wid * b_per_w
        pltpu.sync_copy(idx_hbm.at[pl.ds(base, b_per_w)], idx_v)
        bufs = (rows0, rows1)
        sems = (sem0, sem1)
        pending = pltpu.async_copy(
            table_hbm.at[idx_v.at[pl.ds(0, CH)]], bufs[0], sems[0])
        for c in range(n_ch):
            nxt = c + 1
            nxt_pending = None
            if nxt < n_ch:
                nxt_pending = pltpu.async_copy(
                    table_hbm.at[idx_v.at[pl.ds(nxt * CH, CH)]],
                    bufs[nxt % 2], sems[nxt % 2])
            pending.wait()
            pltpu.sync_copy(bufs[c % 2], out_hbm.at[pl.ds(base + c * CH, CH)])
            pending = nxt_pending

    return k(emb, idx_flat)


_N_PAD = 256


def _tc_body(gat_ref, adj_ref, sent_ref, len_ref,
             wt_ref, bt_ref, w0t_ref, w1t_ref,
             wq_ref, bq_ref, wkt_ref, bk_ref, wvt_ref, bv_ref,
             fcw_p0_ref, fcw_p1_ref, fcw_s_ref, fcb0_ref, lng_ref, lnb_ref,
             fcw1_ref, fcb1_ref,
             attn_ref, logit_ref,
             an_ref, h_ref):
    f32 = jnp.float32

    @pl.when(pl.program_id(0) == 0)
    def _init():
        an_ref[...] = jnp.zeros((_N_REL, _N_PAD, _N_NODE), f32)
        h_ref[...] = jnp.zeros((_C_DIM, _N_REL, _N_PAD), f32)

    g = gat_ref[0]
    x0 = _gelu(jnp.dot(g, wt_ref[...], preferred_element_type=f32)
               + bt_ref[...])
    out_t = x0.T

    for r in range(_N_REL):
        a_r = adj_ref[0, r]
        deg = jnp.maximum(jnp.sum(a_r, axis=0, keepdims=True), 1.0)
        an_ref[r, : _N_NODE, :] = a_r / deg

    def rgcn_layer(out_t, w_ref):
        for r in range(_N_REL):
            h_t = jnp.dot(w_ref[r], out_t, preferred_element_type=f32)
            h_ref[:, r, : _N_NODE] = h_t
        h_flat = h_ref[...].reshape(_C_DIM, _N_REL * _N_PAD)
        an_flat = an_ref[...].reshape(_N_REL * _N_PAD, _N_NODE)
        return _gelu(jnp.dot(h_flat, an_flat, preferred_element_type=f32))

    out_t = rgcn_layer(out_t, w0t_ref)
    out_t = rgcn_layer(out_t, w1t_ref)

    sent = sent_ref[0]
    q = jnp.dot(sent, wq_ref[...], preferred_element_type=f32) + bq_ref[...]
    kk_t = jnp.dot(wkt_ref[...], out_t, preferred_element_type=f32) \
        + bk_ref[...]
    vv_t = jnp.dot(wvt_ref[...], out_t, preferred_element_type=f32) \
        + bv_ref[...]

    length = len_ref[0, 0, 0]
    node_ids = lax.broadcasted_iota(jnp.int32, (1, _N_NODE), 1)
    masked = node_ids >= length

    pooled = []
    attn_rows = []
    for h in range(2):
        q_h = q[:, h * _D_K:(h + 1) * _D_K]
        k_h = kk_t[h * _D_K:(h + 1) * _D_K, :]
        v_h = vv_t[h * _D_K:(h + 1) * _D_K, :]
        score = jnp.dot(q_h, k_h, preferred_element_type=f32) \
            * (1.0 / 7.0710678118654755)
        score = jnp.where(masked, -1e30, score)
        mx = jnp.max(score, axis=1, keepdims=True)
        e = jnp.exp(score - mx)
        attn = e / jnp.sum(e, axis=1, keepdims=True)
        attn_rows.append(attn)
        pooled.append(lax.dot_general(
            attn, v_h, (((1,), (1,)), ((), ())),
            preferred_element_type=f32))

    attn_ref[0] = jnp.concatenate(attn_rows, axis=0)

    f = (jnp.dot(pooled[0], fcw_p0_ref[...], preferred_element_type=f32)
         + jnp.dot(pooled[1], fcw_p1_ref[...], preferred_element_type=f32)
         + jnp.dot(sent, fcw_s_ref[...], preferred_element_type=f32)
         + fcb0_ref[...])
    mu = jnp.mean(f, axis=1, keepdims=True)
    d = f - mu
    v = jnp.mean(d * d, axis=1, keepdims=True)
    y = d * lax.rsqrt(v + 1e-5) * lng_ref[...] + lnb_ref[...]
    hh = _gelu(y)
    logit_ref[0] = jnp.dot(hh, fcw1_ref[...], preferred_element_type=f32) \
        + fcb1_ref[...]


def kernel(sent_vecs, concepts, adj, adj_lengths, emb, Wt, bt, gnn_w0, gnn_w1,
           wq, bq, wk, bk, wv, bv, fc_w0, fc_b0, ln_g, ln_b, fc_w1, fc_b1):
    f32 = jnp.float32
    bs = sent_vecs.shape[0]

    idx_flat = concepts.reshape(-1).astype(jnp.int32)
    gathered = _sc_gather(emb, idx_flat).reshape(bs, _N_NODE, 1024)

    w0t = jnp.transpose(gnn_w0.reshape(_C_DIM, _C_DIM, _N_REL), (2, 1, 0))
    w1t = jnp.transpose(gnn_w1.reshape(_C_DIM, _C_DIM, _N_REL), (2, 1, 0))
    sent3 = sent_vecs.reshape(bs, 1, _SENT_DIM)
    len3 = adj_lengths.reshape(bs, 1, 1).astype(jnp.int32)
    bt_row = bt.reshape(1, _C_DIM)
    bq_row = bq.reshape(1, _C_DIM)
    wkt = wk.T
    wvt = wv.T
    bk_col = bk.reshape(_C_DIM, 1)
    bv_col = bv.reshape(_C_DIM, 1)
    fcw_p0 = fc_w0[:_D_K]
    fcw_p1 = fc_w0[_D_K:2 * _D_K]
    fcw_s = fc_w0[2 * _D_K:]
    fcb0_row = fc_b0.reshape(1, _FC_DIM)
    lng_row = ln_g.reshape(1, _FC_DIM)
    lnb_row = ln_b.reshape(1, _FC_DIM)
    fcb1_row = fc_b1.reshape(1, 1)

    full = lambda *shape: pl.BlockSpec(shape, lambda b: (0,) * len(shape))
    attn_o, logit_o = pl.pallas_call(
        _tc_body,
        grid=(bs,),
        in_specs=[
            pl.BlockSpec((1, _N_NODE, 1024), lambda b: (b, 0, 0)),
            pl.BlockSpec((1, _N_REL, _N_NODE, _N_NODE),
                         lambda b: (b, 0, 0, 0)),
            pl.BlockSpec((1, 1, _SENT_DIM), lambda b: (b, 0, 0)),
            pl.BlockSpec((1, 1, 1), lambda b: (b, 0, 0)),
            full(1024, _C_DIM),
            full(1, _C_DIM),
            full(_N_REL, _C_DIM, _C_DIM),
            full(_N_REL, _C_DIM, _C_DIM),
            full(_SENT_DIM, _C_DIM),
            full(1, _C_DIM),
            full(_C_DIM, _C_DIM),
            full(_C_DIM, 1),
            full(_C_DIM, _C_DIM),
            full(_C_DIM, 1),
            full(_D_K, _FC_DIM),
            full(_D_K, _FC_DIM),
            full(_SENT_DIM, _FC_DIM),
            full(1, _FC_DIM),
            full(1, _FC_DIM),
            full(1, _FC_DIM),
            full(_FC_DIM, 1),
            full(1, 1),
        ],
        out_specs=[
            pl.BlockSpec((1, 2, _N_NODE), lambda b: (b, 0, 0)),
            pl.BlockSpec((1, 1, 1), lambda b: (b, 0, 0)),
        ],
        out_shape=[
            jax.ShapeDtypeStruct((bs, 2, _N_NODE), f32),
            jax.ShapeDtypeStruct((bs, 1, 1), f32),
        ],
        scratch_shapes=[
            pltpu.VMEM((_N_REL, _N_PAD, _N_NODE), f32),
            pltpu.VMEM((_C_DIM, _N_REL, _N_PAD), f32),
        ],
    )(gathered, adj, sent3, len3, Wt, bt_row, w0t, w1t,
      wq, bq_row, wkt, bk_col, wvt, bv_col,
      fcw_p0, fcw_p1, fcw_s, fcb0_row, lng_row, lnb_row, fc_w1, fcb1_row)

    attn = jnp.transpose(attn_o, (1, 0, 2)).reshape(2 * bs, _N_NODE)
    logits = logit_o.reshape(bs, 1)
    return logits, attn


# speedup vs baseline: 7.8535x; 1.6138x over previous
import functools

import jax
import jax.numpy as jnp
from jax import lax
from jax.experimental import pallas as pl
from jax.experimental.pallas import tpu as pltpu
from jax.experimental.pallas import tpu_sc as plsc

_BS, _N_NODE, _N_REL = 32, 200, 34
_C_DIM = 100
_D_K = 50
_SENT_DIM = 1024
_FC_DIM = 200


def _gelu(x):
    return 0.5 * x * (1.0 + lax.erf(x * 0.7071067811865476))


def _sc_gather(emb, idx_flat):
    B, D = idx_flat.shape[0], emb.shape[1]
    NW = 32
    b_per_w = B // NW
    CH = 40
    n_ch = b_per_w // CH
    mesh = plsc.VectorSubcoreMesh(core_axis_name="c", subcore_axis_name="s")

    @functools.partial(
        pl.kernel,
        mesh=mesh,
        out_type=jax.ShapeDtypeStruct((B, D), jnp.float32),
        scratch_types=[
            pltpu.VMEM((b_per_w,), jnp.int32),
            pltpu.VMEM((CH, D), jnp.float32),
            pltpu.VMEM((CH, D), jnp.float32),
            pltpu.SemaphoreType.DMA,
            pltpu.SemaphoreType.DMA,
        ],
    )
    def k(table_hbm, idx_hbm, out_hbm, idx_v, rows0, rows1, sem0, sem1):
        wid = lax.axis_index("s") * 2 + lax.axis_index("c")
        base = wid * b_per_w
        pltpu.sync_copy(idx_hbm.at[pl.ds(base, b_per_w)], idx_v)
        bufs = (rows0, rows1)
        sems = (sem0, sem1)
        pending = pltpu.async_copy(
            table_hbm.at[idx_v.at[pl.ds(0, CH)]], bufs[0], sems[0])
        for c in range(n_ch):
            nxt = c + 1
            nxt_pending = None
            if nxt < n_ch:
                nxt_pending = pltpu.async_copy(
                    table_hbm.at[idx_v.at[pl.ds(nxt * CH, CH)]],
                    bufs[nxt % 2], sems[nxt % 2])
            pending.wait()
            pltpu.sync_copy(bufs[c % 2], out_hbm.at[pl.ds(base + c * CH, CH)])
            pending = nxt_pending

    return k(emb, idx_flat)


_C_PAD = 104


def _tc_body(gat_ref, adj_ref, sent_ref, len_ref,
             wt_ref, bt_ref, w0t_ref, w1t_ref,
             wq_ref, bq_ref, wkt_ref, bk_ref, wvt_ref, bv_ref,
             fcw_p0_ref, fcw_p1_ref, fcw_s_ref, fcb0_ref, lng_ref, lnb_ref,
             fcw1_ref, fcb1_ref,
             attn_ref, logit_ref,
             an_ref, h_ref):
    f32 = jnp.float32

    g = gat_ref[0]
    x0 = _gelu(jnp.dot(g, wt_ref[...], preferred_element_type=f32)
               + bt_ref[...])
    out_t = x0.T

    for r in range(_N_REL):
        a_r = adj_ref[0, r]
        rdeg = 1.0 / jnp.maximum(jnp.sum(a_r, axis=0, keepdims=True), 1.0)
        an_ref[r] = a_r * rdeg

    def rgcn_layer(out_t, w_ref):
        hbig = jnp.dot(w_ref[...], out_t, preferred_element_type=f32)
        h_ref[...] = hbig.reshape(_N_REL, _C_PAD, _N_NODE)
        acc = jnp.zeros((_C_PAD, _N_NODE), f32)
        for r in range(_N_REL):
            acc = acc + jnp.dot(h_ref[r], an_ref[r],
                                preferred_element_type=f32)
        return _gelu(acc)

    out_t = rgcn_layer(out_t, w0t_ref)
    out_t = rgcn_layer(out_t, w1t_ref)

    sent = sent_ref[0]
    q = jnp.dot(sent, wq_ref[...], preferred_element_type=f32) + bq_ref[...]
    kk_t = jnp.dot(wkt_ref[...], out_t, preferred_element_type=f32) \
        + bk_ref[...]
    vv_t = jnp.dot(wvt_ref[...], out_t, preferred_element_type=f32) \
        + bv_ref[...]

    length = len_ref[0, 0, 0]
    node_ids = lax.broadcasted_iota(jnp.int32, (1, _N_NODE), 1)
    masked = node_ids >= length

    pooled = []
    attn_rows = []
    for h in range(2):
        q_h = q[:, h * _D_K:(h + 1) * _D_K]
        k_h = kk_t[h * _D_K:(h + 1) * _D_K, :]
        v_h = vv_t[h * _D_K:(h + 1) * _D_K, :]
        score = jnp.dot(q_h, k_h, preferred_element_type=f32) \
            * (1.0 / 7.0710678118654755)
        score = jnp.where(masked, -1e30, score)
        mx = jnp.max(score, axis=1, keepdims=True)
        e = jnp.exp(score - mx)
        attn = e / jnp.sum(e, axis=1, keepdims=True)
        attn_rows.append(attn)
        pooled.append(lax.dot_general(
            attn, v_h, (((1,), (1,)), ((), ())),
            preferred_element_type=f32))

    attn_ref[0] = jnp.concatenate(attn_rows, axis=0)

    f = (jnp.dot(pooled[0], fcw_p0_ref[...], preferred_element_type=f32)
         + jnp.dot(pooled[1], fcw_p1_ref[...], preferred_element_type=f32)
         + jnp.dot(sent, fcw_s_ref[...], preferred_element_type=f32)
         + fcb0_ref[...])
    mu = jnp.mean(f, axis=1, keepdims=True)
    d = f - mu
    v = jnp.mean(d * d, axis=1, keepdims=True)
    y = d * lax.rsqrt(v + 1e-5) * lng_ref[...] + lnb_ref[...]
    hh = _gelu(y)
    logit_ref[0] = jnp.dot(hh, fcw1_ref[...], preferred_element_type=f32) \
        + fcb1_ref[...]


def kernel(sent_vecs, concepts, adj, adj_lengths, emb, Wt, bt, gnn_w0, gnn_w1,
           wq, bq, wk, bk, wv, bv, fc_w0, fc_b0, ln_g, ln_b, fc_w1, fc_b1):
    f32 = jnp.float32
    bs = sent_vecs.shape[0]

    idx_flat = concepts.reshape(-1).astype(jnp.int32)
    gathered = _sc_gather(emb, idx_flat).reshape(bs, _N_NODE, 1024)

    def stack_w(w):
        wt3 = jnp.transpose(w.reshape(_C_DIM, _C_DIM, _N_REL), (2, 1, 0))
        wt3 = jnp.pad(wt3, ((0, 0), (0, _C_PAD - _C_DIM),
                            (0, _C_PAD - _C_DIM)))
        return wt3.reshape(_N_REL * _C_PAD, _C_PAD)

    w0t = stack_w(gnn_w0)
    w1t = stack_w(gnn_w1)
    wt_p = jnp.pad(Wt, ((0, 0), (0, _C_PAD - _C_DIM)))
    sent3 = sent_vecs.reshape(bs, 1, _SENT_DIM)
    len3 = adj_lengths.reshape(bs, 1, 1).astype(jnp.int32)
    bt_row = jnp.pad(bt.reshape(1, _C_DIM), ((0, 0), (0, _C_PAD - _C_DIM)))
    bq_row = bq.reshape(1, _C_DIM)
    wkt = jnp.pad(wk.T, ((0, 0), (0, _C_PAD - _C_DIM)))
    wvt = jnp.pad(wv.T, ((0, 0), (0, _C_PAD - _C_DIM)))
    bk_col = bk.reshape(_C_DIM, 1)
    bv_col = bv.reshape(_C_DIM, 1)
    fcw_p0 = fc_w0[:_D_K]
    fcw_p1 = fc_w0[_D_K:2 * _D_K]
    fcw_s = fc_w0[2 * _D_K:]
    fcb0_row = fc_b0.reshape(1, _FC_DIM)
    lng_row = ln_g.reshape(1, _FC_DIM)
    lnb_row = ln_b.reshape(1, _FC_DIM)
    fcb1_row = fc_b1.reshape(1, 1)

    full = lambda *shape: pl.BlockSpec(shape, lambda b: (0,) * len(shape))
    attn_o, logit_o = pl.pallas_call(
        _tc_body,
        grid=(bs,),
        in_specs=[
            pl.BlockSpec((1, _N_NODE, 1024), lambda b: (b, 0, 0)),
            pl.BlockSpec((1, _N_REL, _N_NODE, _N_NODE),
                         lambda b: (b, 0, 0, 0)),
            pl.BlockSpec((1, 1, _SENT_DIM), lambda b: (b, 0, 0)),
            pl.BlockSpec((1, 1, 1), lambda b: (b, 0, 0)),
            full(1024, _C_PAD),
            full(1, _C_PAD),
            full(_N_REL * _C_PAD, _C_PAD),
            full(_N_REL * _C_PAD, _C_PAD),
            full(_SENT_DIM, _C_DIM),
            full(1, _C_DIM),
            full(_C_DIM, _C_PAD),
            full(_C_DIM, 1),
            full(_C_DIM, _C_PAD),
            full(_C_DIM, 1),
            full(_D_K, _FC_DIM),
            full(_D_K, _FC_DIM),
            full(_SENT_DIM, _FC_DIM),
            full(1, _FC_DIM),
            full(1, _FC_DIM),
            full(1, _FC_DIM),
            full(_FC_DIM, 1),
            full(1, 1),
        ],
        out_specs=[
            pl.BlockSpec((1, 2, _N_NODE), lambda b: (b, 0, 0)),
            pl.BlockSpec((1, 1, 1), lambda b: (b, 0, 0)),
        ],
        out_shape=[
            jax.ShapeDtypeStruct((bs, 2, _N_NODE), f32),
            jax.ShapeDtypeStruct((bs, 1, 1), f32),
        ],
        scratch_shapes=[
            pltpu.VMEM((_N_REL, _N_NODE, _N_NODE), f32),
            pltpu.VMEM((_N_REL, _C_PAD, _N_NODE), f32),
        ],
    )(gathered, adj, sent3, len3, wt_p, bt_row, w0t, w1t,
      wq, bq_row, wkt, bk_col, wvt, bv_col,
      fcw_p0, fcw_p1, fcw_s, fcb0_row, lng_row, lnb_row, fc_w1, fcb1_row)

    attn = jnp.transpose(attn_o, (1, 0, 2)).reshape(2 * bs, _N_NODE)
    logits = logit_o.reshape(bs, 1)
    return logits, attn


# speedup vs baseline: 7.8598x; 1.0008x over previous
import functools

import jax
import jax.numpy as jnp
from jax import lax
from jax.experimental import pallas as pl
from jax.experimental.pallas import tpu as pltpu
from jax.experimental.pallas import tpu_sc as plsc

_BS, _N_NODE, _N_REL = 32, 200, 34
_C_DIM = 100
_D_K = 50
_SENT_DIM = 1024
_FC_DIM = 200


def _gelu(x):
    return 0.5 * x * (1.0 + lax.erf(x * 0.7071067811865476))


def _sc_gather(emb, idx_flat):
    B, D = idx_flat.shape[0], emb.shape[1]
    NW = 32
    b_per_w = B // NW
    CH = 40
    n_ch = b_per_w // CH
    mesh = plsc.VectorSubcoreMesh(core_axis_name="c", subcore_axis_name="s")

    @functools.partial(
        pl.kernel,
        mesh=mesh,
        out_type=jax.ShapeDtypeStruct((B, D), jnp.float32),
        scratch_types=[
            pltpu.VMEM((b_per_w,), jnp.int32),
            pltpu.VMEM((CH, D), jnp.float32),
            pltpu.VMEM((CH, D), jnp.float32),
            pltpu.SemaphoreType.DMA,
            pltpu.SemaphoreType.DMA,
        ],
    )
    def k(table_hbm, idx_hbm, out_hbm, idx_v, rows0, rows1, sem0, sem1):
        wid = lax.axis_index("s") * 2 + lax.axis_index("c")
        base = wid * b_per_w
        pltpu.sync_copy(idx_hbm.at[pl.ds(base, b_per_w)], idx_v)
        bufs = (rows0, rows1)
        sems = (sem0, sem1)
        pending = pltpu.async_copy(
            table_hbm.at[idx_v.at[pl.ds(0, CH)]], bufs[0], sems[0])
        for c in range(n_ch):
            nxt = c + 1
            nxt_pending = None
            if nxt < n_ch:
                nxt_pending = pltpu.async_copy(
                    table_hbm.at[idx_v.at[pl.ds(nxt * CH, CH)]],
                    bufs[nxt % 2], sems[nxt % 2])
            pending.wait()
            pltpu.sync_copy(bufs[c % 2], out_hbm.at[pl.ds(base + c * CH, CH)])
            pending = nxt_pending

    return k(emb, idx_flat)


_C_PAD = 104


def _tc_body(gat_ref, adj_ref, sent_ref, len_ref,
             wt_ref, bt_ref, w0t_ref, w1t_ref,
             wq_ref, bq_ref, wkt_ref, bk_ref, wvt_ref, bv_ref,
             fcw_p0_ref, fcw_p1_ref, fcw_s_ref, fcb0_ref, lng_ref, lnb_ref,
             fcw1_ref, fcb1_ref,
             attn_ref, logit_ref,
             an_ref, h_ref):
    f32 = jnp.float32

    g = gat_ref[0]
    x0 = _gelu(jnp.dot(g, wt_ref[...], preferred_element_type=f32)
               + bt_ref[...])
    out_t = x0.T

    for r in range(_N_REL):
        a_r = adj_ref[0, r]
        rdeg = 1.0 / jnp.maximum(jnp.sum(a_r, axis=0, keepdims=True), 1.0)
        an_ref[r] = a_r * rdeg

    def rgcn_layer(out_t, w_ref):
        hbig = jnp.dot(w_ref[...], out_t, preferred_element_type=f32)
        h_ref[...] = hbig.reshape(_N_REL, _C_PAD, _N_NODE)
        acc = jnp.zeros((_C_PAD, _N_NODE), f32)
        for r in range(_N_REL):
            acc = acc + jnp.dot(h_ref[r], an_ref[r],
                                preferred_element_type=f32)
        return _gelu(acc)

    out_t = rgcn_layer(out_t, w0t_ref)
    out_t = rgcn_layer(out_t, w1t_ref)

    sent = sent_ref[0]
    q = jnp.dot(sent, wq_ref[...], preferred_element_type=f32) + bq_ref[...]
    kk_t = jnp.dot(wkt_ref[...], out_t, preferred_element_type=f32) \
        + bk_ref[...]
    vv_t = jnp.dot(wvt_ref[...], out_t, preferred_element_type=f32) \
        + bv_ref[...]

    length = len_ref[0, 0, 0]
    node_ids = lax.broadcasted_iota(jnp.int32, (1, _N_NODE), 1)
    masked = node_ids >= length

    pooled = []
    attn_rows = []
    for h in range(2):
        q_h = q[:, h * _D_K:(h + 1) * _D_K]
        k_h = kk_t[h * _D_K:(h + 1) * _D_K, :]
        v_h = vv_t[h * _D_K:(h + 1) * _D_K, :]
        score = jnp.dot(q_h, k_h, preferred_element_type=f32) \
            * (1.0 / 7.0710678118654755)
        score = jnp.where(masked, -1e30, score)
        mx = jnp.max(score, axis=1, keepdims=True)
        e = jnp.exp(score - mx)
        attn = e / jnp.sum(e, axis=1, keepdims=True)
        attn_rows.append(attn)
        pooled.append(lax.dot_general(
            attn, v_h, (((1,), (1,)), ((), ())),
            preferred_element_type=f32))

    attn_ref[0] = jnp.concatenate(attn_rows, axis=0)

    f = (jnp.dot(pooled[0], fcw_p0_ref[...], preferred_element_type=f32)
         + jnp.dot(pooled[1], fcw_p1_ref[...], preferred_element_type=f32)
         + jnp.dot(sent, fcw_s_ref[...], preferred_element_type=f32)
         + fcb0_ref[...])
    mu = jnp.mean(f, axis=1, keepdims=True)
    d = f - mu
    v = jnp.mean(d * d, axis=1, keepdims=True)
    y = d * lax.rsqrt(v + 1e-5) * lng_ref[...] + lnb_ref[...]
    hh = _gelu(y)
    logit_ref[0] = jnp.dot(hh, fcw1_ref[...], preferred_element_type=f32) \
        + fcb1_ref[...]


def kernel(sent_vecs, concepts, adj, adj_lengths, emb, Wt, bt, gnn_w0, gnn_w1,
           wq, bq, wk, bk, wv, bv, fc_w0, fc_b0, ln_g, ln_b, fc_w1, fc_b1):
    f32 = jnp.float32
    bs = sent_vecs.shape[0]

    idx_flat = concepts.reshape(-1).astype(jnp.int32)
    gathered = _sc_gather(emb, idx_flat).reshape(bs, _N_NODE, 1024)

    def stack_w(w):
        wt3 = jnp.transpose(w.reshape(_C_DIM, _C_DIM, _N_REL), (2, 1, 0))
        wt3 = jnp.pad(wt3, ((0, 0), (0, _C_PAD - _C_DIM),
                            (0, _C_PAD - _C_DIM)))
        return wt3.reshape(_N_REL * _C_PAD, _C_PAD)

    w0t = stack_w(gnn_w0)
    w1t = stack_w(gnn_w1)
    wt_p = jnp.pad(Wt, ((0, 0), (0, _C_PAD - _C_DIM)))
    sent3 = sent_vecs.reshape(bs, 1, _SENT_DIM)
    len3 = adj_lengths.reshape(bs, 1, 1).astype(jnp.int32)
    bt_row = jnp.pad(bt.reshape(1, _C_DIM), ((0, 0), (0, _C_PAD - _C_DIM)))
    bq_row = bq.reshape(1, _C_DIM)
    wkt = jnp.pad(wk.T, ((0, 0), (0, _C_PAD - _C_DIM)))
    wvt = jnp.pad(wv.T, ((0, 0), (0, _C_PAD - _C_DIM)))
    bk_col = bk.reshape(_C_DIM, 1)
    bv_col = bv.reshape(_C_DIM, 1)
    fcw_p0 = fc_w0[:_D_K]
    fcw_p1 = fc_w0[_D_K:2 * _D_K]
    fcw_s = fc_w0[2 * _D_K:]
    fcb0_row = fc_b0.reshape(1, _FC_DIM)
    lng_row = ln_g.reshape(1, _FC_DIM)
    lnb_row = ln_b.reshape(1, _FC_DIM)
    fcb1_row = fc_b1.reshape(1, 1)

    full = lambda *shape: pl.BlockSpec(shape, lambda b: (0,) * len(shape))
    attn_o, logit_o = pl.pallas_call(
        _tc_body,
        grid=(bs,),
        in_specs=[
            pl.BlockSpec((1, _N_NODE, 1024), lambda b: (b, 0, 0)),
            pl.BlockSpec((1, _N_REL, _N_NODE, _N_NODE),
                         lambda b: (b, 0, 0, 0)),
            pl.BlockSpec((1, 1, _SENT_DIM), lambda b: (b, 0, 0)),
            pl.BlockSpec((1, 1, 1), lambda b: (b, 0, 0)),
            full(1024, _C_PAD),
            full(1, _C_PAD),
            full(_N_REL * _C_PAD, _C_PAD),
            full(_N_REL * _C_PAD, _C_PAD),
            full(_SENT_DIM, _C_DIM),
            full(1, _C_DIM),
            full(_C_DIM, _C_PAD),
            full(_C_DIM, 1),
            full(_C_DIM, _C_PAD),
            full(_C_DIM, 1),
            full(_D_K, _FC_DIM),
            full(_D_K, _FC_DIM),
            full(_SENT_DIM, _FC_DIM),
            full(1, _FC_DIM),
            full(1, _FC_DIM),
            full(1, _FC_DIM),
            full(_FC_DIM, 1),
            full(1, 1),
        ],
        out_specs=[
            pl.BlockSpec((1, 2, _N_NODE), lambda b: (b, 0, 0)),
            pl.BlockSpec((1, 1, 1), lambda b: (b, 0, 0)),
        ],
        out_shape=[
            jax.ShapeDtypeStruct((bs, 2, _N_NODE), f32),
            jax.ShapeDtypeStruct((bs, 1, 1), f32),
        ],
        scratch_shapes=[
            pltpu.VMEM((_N_REL, _N_NODE, _N_NODE), f32),
            pltpu.VMEM((_N_REL, _C_PAD, _N_NODE), f32),
        ],
        compiler_params=pltpu.CompilerParams(
            dimension_semantics=("parallel",)),
    )(gathered, adj, sent3, len3, wt_p, bt_row, w0t, w1t,
      wq, bq_row, wkt, bk_col, wvt, bv_col,
      fcw_p0, fcw_p1, fcw_s, fcb0_row, lng_row, lnb_row, fc_w1, fcb1_row)

    attn = jnp.transpose(attn_o, (1, 0, 2)).reshape(2 * bs, _N_NODE)
    logits = logit_o.reshape(bs, 1)
    return logits, attn


# speedup vs baseline: 9.8174x; 1.2491x over previous
import functools

import jax
import jax.numpy as jnp
from jax import lax
from jax.experimental import pallas as pl
from jax.experimental.pallas import tpu as pltpu
from jax.experimental.pallas import tpu_sc as plsc

_BS, _N_NODE, _N_REL = 32, 200, 34
_C_DIM = 100
_D_K = 50
_SENT_DIM = 1024
_FC_DIM = 200


def _gelu(x):
    return 0.5 * x * (1.0 + lax.erf(x * 0.7071067811865476))


def _sc_gather(emb, idx_flat):
    B, D = idx_flat.shape[0], emb.shape[1]
    NW = 32
    b_per_w = B // NW
    CH = 40
    n_ch = b_per_w // CH
    mesh = plsc.VectorSubcoreMesh(core_axis_name="c", subcore_axis_name="s")

    @functools.partial(
        pl.kernel,
        mesh=mesh,
        out_type=jax.ShapeDtypeStruct((B, D), jnp.float32),
        scratch_types=[
            pltpu.VMEM((b_per_w,), jnp.int32),
            pltpu.VMEM((CH, D), jnp.float32),
            pltpu.VMEM((CH, D), jnp.float32),
            pltpu.SemaphoreType.DMA,
            pltpu.SemaphoreType.DMA,
        ],
    )
    def k(table_hbm, idx_hbm, out_hbm, idx_v, rows0, rows1, sem0, sem1):
        wid = lax.axis_index("s") * 2 + lax.axis_index("c")
        base = wid * b_per_w
        pltpu.sync_copy(idx_hbm.at[pl.ds(base, b_per_w)], idx_v)
        bufs = (rows0, rows1)
        sems = (sem0, sem1)
        pending = pltpu.async_copy(
            table_hbm.at[idx_v.at[pl.ds(0, CH)]], bufs[0], sems[0])
        for c in range(n_ch):
            nxt = c + 1
            nxt_pending = None
            if nxt < n_ch:
                nxt_pending = pltpu.async_copy(
                    table_hbm.at[idx_v.at[pl.ds(nxt * CH, CH)]],
                    bufs[nxt % 2], sems[nxt % 2])
            pending.wait()
            pltpu.sync_copy(bufs[c % 2], out_hbm.at[pl.ds(base + c * CH, CH)])
            pending = nxt_pending

    return k(emb, idx_flat)


_C_PAD = 104


def _tc_body(gat_ref, adj_ref, sent_ref, len_ref,
             wt_ref, bt_ref, w0t_ref, w1t_ref,
             wq_ref, bq_ref, wkt_ref, bk_ref, wvt_ref, bv_ref,
             fcw_p0_ref, fcw_p1_ref, fcw_s_ref, fcb0_ref, lng_ref, lnb_ref,
             fcw1_ref, fcb1_ref, sal_ref,
             attn_ref, logit_ref,
             an_ref, h_ref, vv_ref, sc_ref):
    f32 = jnp.float32

    g = gat_ref[0]
    x0 = _gelu(jnp.dot(g, wt_ref[...], preferred_element_type=f32)
               + bt_ref[...])
    out_t = x0.T

    for r in range(_N_REL):
        a_r = adj_ref[0, r]
        rdeg = 1.0 / jnp.maximum(jnp.sum(a_r, axis=0, keepdims=True), 1.0)
        an_ref[r] = (a_r * rdeg).astype(jnp.bfloat16)

    def rgcn_layer(out_t, w_ref):
        hbig = jnp.dot(w_ref[...], out_t, preferred_element_type=f32)
        h_ref[...] = hbig.reshape(_N_REL, _C_PAD, _N_NODE).astype(jnp.bfloat16)
        acc = jnp.zeros((_C_PAD, _N_NODE), f32)
        for r in range(_N_REL):
            acc = acc + jnp.dot(h_ref[r], an_ref[r],
                                preferred_element_type=f32)
        return _gelu(acc)

    out_t = rgcn_layer(out_t, w0t_ref)
    out_t = rgcn_layer(out_t, w1t_ref)

    b = pl.program_id(0)
    sent = sent_ref[0]
    q = jnp.dot(sent, wq_ref[...], preferred_element_type=f32) + bq_ref[...]
    kk_t = jnp.dot(wkt_ref[...], out_t, preferred_element_type=f32) \
        + bk_ref[...]
    vv_t = jnp.dot(wvt_ref[...], out_t, preferred_element_type=f32) \
        + bv_ref[...]
    vv_ref[b] = vv_t
    scs = []
    for h in range(2):
        q_h = q[:, h * _D_K:(h + 1) * _D_K]
        k_h = kk_t[h * _D_K:(h + 1) * _D_K, :]
        scs.append(jnp.dot(q_h, k_h, preferred_element_type=f32)
                   * (1.0 / 7.0710678118654755))
    sc_ref[b] = jnp.concatenate(scs, axis=0)

    @pl.when(b == pl.num_programs(0) - 1)
    def _tail():
        scores = sc_ref[...]
        lens = len_ref[...]
        node_ids = lax.broadcasted_iota(jnp.int32, (1, 1, _N_NODE), 2)
        s = jnp.where(node_ids >= lens, -1e30, scores)
        mx = jnp.max(s, axis=2, keepdims=True)
        e = jnp.exp(s - mx)
        attn_all = e / jnp.sum(e, axis=2, keepdims=True)
        attn_ref[...] = attn_all
        vv = vv_ref[...]
        sent_all = sal_ref[...]
        pooled = []
        for h in range(2):
            ah = attn_all[:, h:h + 1, :]
            vh = vv[:, h * _D_K:(h + 1) * _D_K, :]
            pooled.append(jnp.sum(vh * ah, axis=2))
        f = (jnp.dot(pooled[0], fcw_p0_ref[...], preferred_element_type=f32)
             + jnp.dot(pooled[1], fcw_p1_ref[...], preferred_element_type=f32)
             + jnp.dot(sent_all, fcw_s_ref[...], preferred_element_type=f32)
             + fcb0_ref[...])
        mu = jnp.mean(f, axis=1, keepdims=True)
        d = f - mu
        v = jnp.mean(d * d, axis=1, keepdims=True)
        y = d * lax.rsqrt(v + 1e-5) * lng_ref[...] + lnb_ref[...]
        hh = _gelu(y)
        logit_ref[...] = jnp.dot(hh, fcw1_ref[...],
                                 preferred_element_type=f32) + fcb1_ref[...]


def kernel(sent_vecs, concepts, adj, adj_lengths, emb, Wt, bt, gnn_w0, gnn_w1,
           wq, bq, wk, bk, wv, bv, fc_w0, fc_b0, ln_g, ln_b, fc_w1, fc_b1):
    f32 = jnp.float32
    bs = sent_vecs.shape[0]

    idx_flat = concepts.reshape(-1).astype(jnp.int32)
    gathered = _sc_gather(emb, idx_flat).reshape(bs, _N_NODE, 1024)

    def stack_w(w):
        wt3 = jnp.transpose(w.reshape(_C_DIM, _C_DIM, _N_REL), (2, 1, 0))
        wt3 = jnp.pad(wt3, ((0, 0), (0, _C_PAD - _C_DIM),
                            (0, _C_PAD - _C_DIM)))
        return wt3.reshape(_N_REL * _C_PAD, _C_PAD)

    w0t = stack_w(gnn_w0)
    w1t = stack_w(gnn_w1)
    wt_p = jnp.pad(Wt, ((0, 0), (0, _C_PAD - _C_DIM)))
    sent3 = sent_vecs.reshape(bs, 1, _SENT_DIM)
    len3 = adj_lengths.reshape(bs, 1, 1).astype(jnp.int32)
    bt_row = jnp.pad(bt.reshape(1, _C_DIM), ((0, 0), (0, _C_PAD - _C_DIM)))
    bq_row = bq.reshape(1, _C_DIM)
    wkt = jnp.pad(wk.T, ((0, 0), (0, _C_PAD - _C_DIM)))
    wvt = jnp.pad(wv.T, ((0, 0), (0, _C_PAD - _C_DIM)))
    bk_col = bk.reshape(_C_DIM, 1)
    bv_col = bv.reshape(_C_DIM, 1)
    fcw_p0 = fc_w0[:_D_K]
    fcw_p1 = fc_w0[_D_K:2 * _D_K]
    fcw_s = fc_w0[2 * _D_K:]
    fcb0_row = fc_b0.reshape(1, _FC_DIM)
    lng_row = ln_g.reshape(1, _FC_DIM)
    lnb_row = ln_b.reshape(1, _FC_DIM)
    fcb1_row = fc_b1.reshape(1, 1)

    full = lambda *shape: pl.BlockSpec(shape, lambda b: (0,) * len(shape))
    attn_o, logit_o = pl.pallas_call(
        _tc_body,
        grid=(bs,),
        in_specs=[
            pl.BlockSpec((1, _N_NODE, 1024), lambda b: (b, 0, 0)),
            pl.BlockSpec((1, _N_REL, _N_NODE, _N_NODE),
                         lambda b: (b, 0, 0, 0)),
            pl.BlockSpec((1, 1, _SENT_DIM), lambda b: (b, 0, 0)),
            full(bs, 1, 1),
            full(1024, _C_PAD),
            full(1, _C_PAD),
            full(_N_REL * _C_PAD, _C_PAD),
            full(_N_REL * _C_PAD, _C_PAD),
            full(_SENT_DIM, _C_DIM),
            full(1, _C_DIM),
            full(_C_DIM, _C_PAD),
            full(_C_DIM, 1),
            full(_C_DIM, _C_PAD),
            full(_C_DIM, 1),
            full(_D_K, _FC_DIM),
            full(_D_K, _FC_DIM),
            full(_SENT_DIM, _FC_DIM),
            full(1, _FC_DIM),
            full(1, _FC_DIM),
            full(1, _FC_DIM),
            full(_FC_DIM, 1),
            full(1, 1),
            full(bs, _SENT_DIM),
        ],
        out_specs=[
            pl.BlockSpec((bs, 2, _N_NODE), lambda b: (0, 0, 0)),
            pl.BlockSpec((bs, 1), lambda b: (0, 0)),
        ],
        out_shape=[
            jax.ShapeDtypeStruct((bs, 2, _N_NODE), f32),
            jax.ShapeDtypeStruct((bs, 1), f32),
        ],
        scratch_shapes=[
            pltpu.VMEM((_N_REL, _N_NODE, _N_NODE), jnp.bfloat16),
            pltpu.VMEM((_N_REL, _C_PAD, _N_NODE), jnp.bfloat16),
            pltpu.VMEM((bs, _C_DIM, _N_NODE), f32),
            pltpu.VMEM((bs, 2, _N_NODE), f32),
        ],
        compiler_params=pltpu.CompilerParams(
            dimension_semantics=("arbitrary",)),
    )(gathered, adj, sent3, len3, wt_p, bt_row, w0t, w1t,
      wq, bq_row, wkt, bk_col, wvt, bv_col,
      fcw_p0, fcw_p1, fcw_s, fcb0_row, lng_row, lnb_row, fc_w1, fcb1_row,
      sent_vecs)

    attn = jnp.transpose(attn_o, (1, 0, 2)).reshape(2 * bs, _N_NODE)
    logits = logit_o
    return logits, attn


# speedup vs baseline: 9.8505x; 1.0034x over previous
import functools

import jax
import jax.numpy as jnp
from jax import lax
from jax.experimental import pallas as pl
from jax.experimental.pallas import tpu as pltpu
from jax.experimental.pallas import tpu_sc as plsc

_BS, _N_NODE, _N_REL = 32, 200, 34
_C_DIM = 100
_D_K = 50
_SENT_DIM = 1024
_FC_DIM = 200


def _gelu(x):
    return 0.5 * x * (1.0 + lax.erf(x * 0.7071067811865476))


def _sc_gather(emb, idx_flat):
    B, D = idx_flat.shape[0], emb.shape[1]
    NW = 32
    b_per_w = B // NW
    CH = 40
    n_ch = b_per_w // CH
    mesh = plsc.VectorSubcoreMesh(core_axis_name="c", subcore_axis_name="s")

    NBUF = 3

    @functools.partial(
        pl.kernel,
        mesh=mesh,
        out_type=jax.ShapeDtypeStruct((B, D), jnp.float32),
        scratch_types=[
            pltpu.VMEM((b_per_w,), jnp.int32),
        ] + [pltpu.VMEM((CH, D), jnp.float32) for _ in range(NBUF)]
          + [pltpu.SemaphoreType.DMA for _ in range(2 * NBUF)],
    )
    def k(table_hbm, idx_hbm, out_hbm, idx_v, *bufs_sems):
        bufs = bufs_sems[:NBUF]
        gsems = bufs_sems[NBUF:2 * NBUF]
        wsems = bufs_sems[2 * NBUF:]
        wid = lax.axis_index("s") * 2 + lax.axis_index("c")
        base = wid * b_per_w
        pltpu.sync_copy(idx_hbm.at[pl.ds(base, b_per_w)], idx_v)
        pend_g = [None] * NBUF
        pend_w = [None] * NBUF
        for c in range(min(NBUF, n_ch)):
            pend_g[c] = pltpu.async_copy(
                table_hbm.at[idx_v.at[pl.ds(c * CH, CH)]],
                bufs[c], gsems[c])
        for c in range(n_ch):
            s = c % NBUF
            pend_g[s].wait()
            pend_w[s] = pltpu.async_copy(
                bufs[s], out_hbm.at[pl.ds(base + c * CH, CH)], wsems[s])
            nxt = c + NBUF
            if nxt < n_ch:
                pend_w[s].wait()
                pend_w[s] = None
                pend_g[s] = pltpu.async_copy(
                    table_hbm.at[idx_v.at[pl.ds(nxt * CH, CH)]],
                    bufs[s], gsems[s])
        for pw in pend_w:
            if pw is not None:
                pw.wait()

    return k(emb, idx_flat)


_C_PAD = 104


def _tc_body(gat_ref, adj_ref, sent_ref, len_ref,
             wt_ref, bt_ref, w0t_ref, w1t_ref,
             wq_ref, bq_ref, wkt_ref, bk_ref, wvt_ref, bv_ref,
             fcw_p0_ref, fcw_p1_ref, fcw_s_ref, fcb0_ref, lng_ref, lnb_ref,
             fcw1_ref, fcb1_ref, sal_ref,
             attn_ref, logit_ref,
             an_ref, h_ref, vv_ref, sc_ref):
    f32 = jnp.float32

    g = gat_ref[0]
    x0 = _gelu(jnp.dot(g, wt_ref[...], preferred_element_type=f32)
               + bt_ref[...])
    out_t = x0.T

    for r in range(_N_REL):
        a_r = adj_ref[0, r]
        rdeg = 1.0 / jnp.maximum(jnp.sum(a_r, axis=0, keepdims=True), 1.0)
        an_ref[r] = (a_r * rdeg).astype(jnp.bfloat16)

    def rgcn_layer(out_t, w_ref):
        hbig = jnp.dot(w_ref[...], out_t, preferred_element_type=f32)
        h_ref[...] = hbig.reshape(_N_REL, _C_PAD, _N_NODE).astype(jnp.bfloat16)
        acc = jnp.zeros((_C_PAD, _N_NODE), f32)
        for r in range(_N_REL):
            acc = acc + jnp.dot(h_ref[r], an_ref[r],
                                preferred_element_type=f32)
        return _gelu(acc)

    out_t = rgcn_layer(out_t, w0t_ref)
    out_t = rgcn_layer(out_t, w1t_ref)

    b = pl.program_id(0)
    sent = sent_ref[0]
    q = jnp.dot(sent, wq_ref[...], preferred_element_type=f32) + bq_ref[...]
    kk_t = jnp.dot(wkt_ref[...], out_t, preferred_element_type=f32) \
        + bk_ref[...]
    vv_t = jnp.dot(wvt_ref[...], out_t, preferred_element_type=f32) \
        + bv_ref[...]
    vv_ref[b] = vv_t
    scs = []
    for h in range(2):
        q_h = q[:, h * _D_K:(h + 1) * _D_K]
        k_h = kk_t[h * _D_K:(h + 1) * _D_K, :]
        scs.append(jnp.dot(q_h, k_h, preferred_element_type=f32)
                   * (1.0 / 7.0710678118654755))
    sc_ref[b] = jnp.concatenate(scs, axis=0)

    @pl.when(b == pl.num_programs(0) - 1)
    def _tail():
        scores = sc_ref[...]
        lens = len_ref[...]
        node_ids = lax.broadcasted_iota(jnp.int32, (1, 1, _N_NODE), 2)
        s = jnp.where(node_ids >= lens, -1e30, scores)
        mx = jnp.max(s, axis=2, keepdims=True)
        e = jnp.exp(s - mx)
        attn_all = e / jnp.sum(e, axis=2, keepdims=True)
        attn_ref[...] = attn_all
        vv = vv_ref[...]
        sent_all = sal_ref[...]
        pooled = []
        for h in range(2):
            ah = attn_all[:, h:h + 1, :]
            vh = vv[:, h * _D_K:(h + 1) * _D_K, :]
            pooled.append(jnp.sum(vh * ah, axis=2))
        f = (jnp.dot(pooled[0], fcw_p0_ref[...], preferred_element_type=f32)
             + jnp.dot(pooled[1], fcw_p1_ref[...], preferred_element_type=f32)
             + jnp.dot(sent_all, fcw_s_ref[...], preferred_element_type=f32)
             + fcb0_ref[...])
        mu = jnp.mean(f, axis=1, keepdims=True)
        d = f - mu
        v = jnp.mean(d * d, axis=1, keepdims=True)
        y = d * lax.rsqrt(v + 1e-5) * lng_ref[...] + lnb_ref[...]
        hh = _gelu(y)
        logit_ref[...] = jnp.dot(hh, fcw1_ref[...],
                                 preferred_element_type=f32) + fcb1_ref[...]


def kernel(sent_vecs, concepts, adj, adj_lengths, emb, Wt, bt, gnn_w0, gnn_w1,
           wq, bq, wk, bk, wv, bv, fc_w0, fc_b0, ln_g, ln_b, fc_w1, fc_b1):
    f32 = jnp.float32
    bs = sent_vecs.shape[0]

    idx_flat = concepts.reshape(-1).astype(jnp.int32)
    gathered = _sc_gather(emb, idx_flat).reshape(bs, _N_NODE, 1024)

    def stack_w(w):
        wt3 = jnp.transpose(w.reshape(_C_DIM, _C_DIM, _N_REL), (2, 1, 0))
        wt3 = jnp.pad(wt3, ((0, 0), (0, _C_PAD - _C_DIM),
                            (0, _C_PAD - _C_DIM)))
        return wt3.reshape(_N_REL * _C_PAD, _C_PAD)

    w0t = stack_w(gnn_w0)
    w1t = stack_w(gnn_w1)
    wt_p = jnp.pad(Wt, ((0, 0), (0, _C_PAD - _C_DIM)))
    sent3 = sent_vecs.reshape(bs, 1, _SENT_DIM)
    len3 = adj_lengths.reshape(bs, 1, 1).astype(jnp.int32)
    bt_row = jnp.pad(bt.reshape(1, _C_DIM), ((0, 0), (0, _C_PAD - _C_DIM)))
    bq_row = bq.reshape(1, _C_DIM)
    wkt = jnp.pad(wk.T, ((0, 0), (0, _C_PAD - _C_DIM)))
    wvt = jnp.pad(wv.T, ((0, 0), (0, _C_PAD - _C_DIM)))
    bk_col = bk.reshape(_C_DIM, 1)
    bv_col = bv.reshape(_C_DIM, 1)
    fcw_p0 = fc_w0[:_D_K]
    fcw_p1 = fc_w0[_D_K:2 * _D_K]
    fcw_s = fc_w0[2 * _D_K:]
    fcb0_row = fc_b0.reshape(1, _FC_DIM)
    lng_row = ln_g.reshape(1, _FC_DIM)
    lnb_row = ln_b.reshape(1, _FC_DIM)
    fcb1_row = fc_b1.reshape(1, 1)

    full = lambda *shape: pl.BlockSpec(shape, lambda b: (0,) * len(shape))
    attn_o, logit_o = pl.pallas_call(
        _tc_body,
        grid=(bs,),
        in_specs=[
            pl.BlockSpec((1, _N_NODE, 1024), lambda b: (b, 0, 0)),
            pl.BlockSpec((1, _N_REL, _N_NODE, _N_NODE),
                         lambda b: (b, 0, 0, 0)),
            pl.BlockSpec((1, 1, _SENT_DIM), lambda b: (b, 0, 0)),
            full(bs, 1, 1),
            full(1024, _C_PAD),
            full(1, _C_PAD),
            full(_N_REL * _C_PAD, _C_PAD),
            full(_N_REL * _C_PAD, _C_PAD),
            full(_SENT_DIM, _C_DIM),
            full(1, _C_DIM),
            full(_C_DIM, _C_PAD),
            full(_C_DIM, 1),
            full(_C_DIM, _C_PAD),
            full(_C_DIM, 1),
            full(_D_K, _FC_DIM),
            full(_D_K, _FC_DIM),
            full(_SENT_DIM, _FC_DIM),
            full(1, _FC_DIM),
            full(1, _FC_DIM),
            full(1, _FC_DIM),
            full(_FC_DIM, 1),
            full(1, 1),
            full(bs, _SENT_DIM),
        ],
        out_specs=[
            pl.BlockSpec((bs, 2, _N_NODE), lambda b: (0, 0, 0)),
            pl.BlockSpec((bs, 1), lambda b: (0, 0)),
        ],
        out_shape=[
            jax.ShapeDtypeStruct((bs, 2, _N_NODE), f32),
            jax.ShapeDtypeStruct((bs, 1), f32),
        ],
        scratch_shapes=[
            pltpu.VMEM((_N_REL, _N_NODE, _N_NODE), jnp.bfloat16),
            pltpu.VMEM((_N_REL, _C_PAD, _N_NODE), jnp.bfloat16),
            pltpu.VMEM((bs, _C_DIM, _N_NODE), f32),
            pltpu.VMEM((bs, 2, _N_NODE), f32),
        ],
        compiler_params=pltpu.CompilerParams(
            dimension_semantics=("arbitrary",)),
    )(gathered, adj, sent3, len3, wt_p, bt_row, w0t, w1t,
      wq, bq_row, wkt, bk_col, wvt, bv_col,
      fcw_p0, fcw_p1, fcw_s, fcb0_row, lng_row, lnb_row, fc_w1, fcb1_row,
      sent_vecs)

    attn = jnp.transpose(attn_o, (1, 0, 2)).reshape(2 * bs, _N_NODE)
    logits = logit_o
    return logits, attn
